# baseline (speedup 1.0000x reference)
import functools

import jax
import jax.numpy as jnp
from jax import lax
from jax.experimental import pallas as pl
from jax.experimental.pallas import tpu as pltpu
from jax.experimental.pallas import tpu_sc as plsc

NUM_TOKENS = 8192
MASK_ID = 3
BS, C, N = 4, 384, 1024

L = 16
NC, NS = 2, 16
NW = NC * NS
ROWS = BS * C
RPW = ROWS // NW
WPB = NW // BS
NPT = N // NS
CCHUNKS = C // L
ZROWS = 4
NZ = RPW // ZROWS
BUFW = ZROWS * NUM_TOKENS


def _all_lanes(x):
    return plsc.all_reduce_population_count(x)[0] == L


def _no_lanes(x):
    return plsc.all_reduce_population_count(x)[0] == 0


def _sc_body(in_hbm, ds_hbm, out_hbm,
             in_v, ds_v, dsrow_v, buf0, buf1, w_local, cnt_v, w_all,
             cnt_all, ns_v, w_sh, cnt_sh, zsem0, zsem1):
    cid = lax.axis_index("c")
    sid = lax.axis_index("s")
    wid = cid * NS + sid
    iota = lax.iota(jnp.int32, L)
    bufs = (buf0, buf1)
    sems = (zsem0, zsem1)

    pltpu.sync_copy(in_hbm, in_v.at[pl.ds(0, BS * C)])
    pltpu.sync_copy(ds_hbm.at[pl.ds(sid * NPT * C, NPT * C)], ds_v)

    row_base = wid * RPW
    b = wid // WPB
    base_c = (wid % WPB) * RPW
    tok_base = b * C + base_c
    ones_vec = jnp.ones((L,), jnp.float32)
    zero_vec = jnp.zeros((L,), jnp.float32)
    punch_mask = iota < ZROWS

    def _holes(k):
        toks = in_v[pl.ds(tok_base + k * ZROWS, L)]
        return iota * NUM_TOKENS + toks

    def _fire(k):
        buf = bufs[k % 2]
        plsc.store_scatter(buf, [_holes(k)], ones_vec, mask=punch_mask)
        dst = out_hbm.at[pl.ds((row_base + k * ZROWS) * NUM_TOKENS, BUFW)]
        return pltpu.async_copy(buf, dst, sems[k % 2])

    def _drain(k, cp):
        cp.wait()
        buf = bufs[k % 2]
        plsc.store_scatter(buf, [_holes(k)], zero_vec, mask=punch_mask)

    def _zloop0(i, _):
        buf0[pl.ds(i * L, L)] = jnp.zeros((L,), jnp.float32)
        return 0
    lax.fori_loop(0, BUFW // L, _zloop0, 0)
    zcopies = [_fire(0)]

    def _zloop1(i, _):
        buf1[pl.ds(i * L, L)] = jnp.zeros((L,), jnp.float32)
        return 0
    lax.fori_loop(0, BUFW // L, _zloop1, 0)
    zcopies.append(_fire(1))

    in0 = [in_v[pl.ds(bb * C, L)] for bb in range(BS)]
    wild0 = [v == MASK_ID for v in in0]

    def _gbody(g, cnts):
        nvec = g * L + iota
        cands = [iota == iota for _ in range(BS)]
        for t in range(8):
            colv = plsc.load_gather(ds_v, [nvec * C + t])
            for bb in range(BS):
                tok_t = in0[bb][t]
                cands[bb] = cands[bb] & ((colv == tok_t) | (tok_t == MASK_ID))
        comb = cands[0]
        for bb in range(1, BS):
            comb = comb | cands[bb]

        def _slow(g=g, cnts=cnts):
            def _lbody(l, inner):
                vecs, cnts = inner
                n = g * L + l
                ds0 = ds_v[pl.ds(n * C, L)]
                new_vecs, new_cnts = [], []
                for bb in range(BS):
                    ok0 = (ds0 == in0[bb]) | wild0[bb]

                    def _full(bb=bb, n=n, ok0=ok0):
                        def _cbody(t, acc):
                            dsv = ds_v[pl.ds(n * C + t * L, L)]
                            inv = in_v[pl.ds(bb * C + t * L, L)]
                            return acc & ((dsv == inv) | (inv == MASK_ID))
                        acc = lax.fori_loop(1, CCHUNKS, _cbody, ok0)
                        return jnp.where(_all_lanes(acc), jnp.float32(1.0),
                                         jnp.float32(0.0))

                    flag = lax.cond(_all_lanes(ok0), _full,
                                    lambda: jnp.float32(0.0))
                    new_vecs.append(jnp.where(iota == l, flag, vecs[bb]))
                    new_cnts.append(cnts[bb] + flag)
                return tuple(new_vecs), tuple(new_cnts)

            vecs, cnts2 = lax.fori_loop(
                0, L, _lbody, (tuple(zero_vec for _ in range(BS)), cnts))
            return vecs + cnts2

        def _fast(cnts=cnts):
            return tuple(zero_vec for _ in range(BS)) + cnts

        res = lax.cond(_no_lanes(comb), _fast, _slow)
        vecs, cnts = res[:BS], res[BS:]
        for bb in range(BS):
            w_local[pl.ds(bb * NPT + g * L, L)] = vecs[bb]
        return cnts

    cnts = lax.fori_loop(0, NPT // L, _gbody,
                         tuple(jnp.float32(0.0) for _ in range(BS)))

    cv = zero_vec
    for bb in range(BS):
        cv = jnp.where(iota == bb, cnts[bb], cv)
    cnt_v[...] = cv

    pltpu.sync_copy(w_local, w_sh.at[pl.ds(sid * (BS * NPT), BS * NPT)])
    pltpu.sync_copy(cnt_v, cnt_sh.at[pl.ds(sid * L, L)])

    for k in range(2, NZ):
        _drain(k - 2, zcopies[k - 2])
        zcopies.append(_fire(k))
    _drain(NZ - 2, zcopies[NZ - 2])
    _drain(NZ - 1, zcopies[NZ - 1])

    plsc.subcore_barrier()
    pltpu.sync_copy(cnt_sh, cnt_all)
    ns = cnt_all[pl.ds(0, L)]
    for s in range(1, NS):
        ns = ns + cnt_all[pl.ds(s * L, L)]
    ns_v[...] = ns

    my_ns = ns[BS - 1]
    for bb in range(BS - 1):
        my_ns = jnp.where(b == bb, ns[bb], my_ns)

    @pl.when(my_ns > 0.0)
    def _match():
        inv_v = ones_vec / jnp.full((L,), my_ns, jnp.float32)
        pltpu.sync_copy(w_sh, w_all)

        def _blk(blk, _):
            def _acc(j, _2):
                pltpu.sync_copy(ds_hbm.at[pl.ds(j * L * C, L * C)], dsrow_v)
                off = ((j // (NPT // L)) * (BS * NPT) + b * NPT
                       + (j % (NPT // L)) * L)
                msk = w_all[pl.ds(off, L)] > 0.5
                for r in range(ZROWS):
                    col = base_c + blk * ZROWS + r
                    toks = plsc.load_gather(dsrow_v, [iota * C + col])
                    plsc.addupdate_scatter(buf0, [toks + r * NUM_TOKENS],
                                           inv_v, mask=msk)
                return 0
            lax.fori_loop(0, N // L, _acc, 0)

            pltpu.sync_copy(
                buf0,
                out_hbm.at[pl.ds((row_base + blk * ZROWS) * NUM_TOKENS,
                                 BUFW)])

            def _z2(i, _2):
                buf0[pl.ds(i * L, L)] = jnp.zeros((L,), jnp.float32)
                return 0
            lax.fori_loop(0, BUFW // L, _z2, 0)
            return 0

        lax.fori_loop(0, NZ, _blk, 0)


@jax.jit
def _impl(input_tokens, dataset_tokens):
    mesh = plsc.VectorSubcoreMesh(core_axis_name="c", subcore_axis_name="s")
    run = functools.partial(
        pl.kernel,
        mesh=mesh,
        compiler_params=pltpu.CompilerParams(needs_layout_passes=False),
        out_type=jax.ShapeDtypeStruct((ROWS * NUM_TOKENS,), jnp.float32),
        scratch_types=[
            pltpu.VMEM((BS * C + L,), jnp.int32),
            pltpu.VMEM((NPT * C,), jnp.int32),
            pltpu.VMEM((L * C,), jnp.int32),
            pltpu.VMEM((BUFW,), jnp.float32),
            pltpu.VMEM((BUFW,), jnp.float32),
            pltpu.VMEM((BS * NPT,), jnp.float32),
            pltpu.VMEM((L,), jnp.float32),
            pltpu.VMEM((NS * BS * NPT,), jnp.float32),
            pltpu.VMEM((NS * L,), jnp.float32),
            pltpu.VMEM((L,), jnp.float32),
            pltpu.VMEM_SHARED((NS * BS * NPT,), jnp.float32),
            pltpu.VMEM_SHARED((NS * L,), jnp.float32),
            pltpu.SemaphoreType.DMA,
            pltpu.SemaphoreType.DMA,
        ],
    )(_sc_body)
    flat = run(input_tokens.reshape(-1), dataset_tokens.reshape(-1))
    return flat.reshape(BS, C, NUM_TOKENS)


def kernel(input_tokens, dataset_tokens, t):
    del t
    return _impl(input_tokens, dataset_tokens)


# speedup vs baseline: 2.6399x; 2.6399x over previous
import functools

import jax
import jax.numpy as jnp
from jax import lax
from jax.experimental import pallas as pl
from jax.experimental.pallas import tpu as pltpu
from jax.experimental.pallas import tpu_sc as plsc

NUM_TOKENS = 8192
MASK_ID = 3
BS, C, N = 4, 384, 1024

L = 16
NC, NS = 2, 16
NW = NC * NS
ROWS = BS * C
RPW = ROWS // NW
WPB = NW // BS
NPT = N // NS
CCHUNKS = C // L
ZROWS = 8
NZ = RPW // ZROWS
SCREEN = 8


def _all_lanes(x):
    return plsc.all_reduce_population_count(x)[0] == L


def _no_lanes(x):
    return plsc.all_reduce_population_count(x)[0] == 0


def _sc_body(in_hbm, ds_hbm, out_hbm,
             in_v, ds_v, dsrow_v, buf, w_local, cnt_v, w_all,
             cnt_all, ns_v, w_sh, cnt_sh, sem):
    cid = lax.axis_index("c")
    sid = lax.axis_index("s")
    wid = cid * NS + sid
    iota = lax.iota(jnp.int32, L)

    pltpu.sync_copy(in_hbm, in_v.at[pl.ds(0, BS * C)])
    pltpu.sync_copy(ds_hbm.at[pl.ds(sid * NPT * C, NPT * C)], ds_v)

    def _zloop(i, _):
        for r in range(ZROWS):
            buf[r, pl.ds(i * L, L)] = jnp.zeros((L,), jnp.float32)
        return 0
    lax.fori_loop(0, NUM_TOKENS // L, _zloop, 0)

    row_base = wid * RPW
    b = wid // WPB
    base_c = (wid % WPB) * RPW
    tok_base = b * C + base_c
    ones_vec = jnp.ones((L,), jnp.float32)
    zero_vec = jnp.zeros((L,), jnp.float32)
    punch_mask = iota < ZROWS

    def _holes(k):
        return in_v[pl.ds(tok_base + k * ZROWS, L)]

    def _fire(k):
        plsc.store_scatter(buf, [iota, _holes(k)], ones_vec, mask=punch_mask)
        dst = out_hbm.at[pl.ds(row_base + k * ZROWS, ZROWS)]
        return pltpu.async_copy(buf, dst, sem)

    def _drain(k, cp):
        cp.wait()
        plsc.store_scatter(buf, [iota, _holes(k)], zero_vec, mask=punch_mask)

    cp = _fire(0)

    in0 = [in_v[pl.ds(bb * C, L)] for bb in range(BS)]
    wild0 = [v == MASK_ID for v in in0]

    def _gbody(g, cnts):
        nvec = g * L + iota
        cands = [iota == iota for _ in range(BS)]
        for t in range(SCREEN):
            colv = plsc.load_gather(ds_v, [nvec * C + t])
            for bb in range(BS):
                tok_t = in0[bb][t]
                cands[bb] = cands[bb] & ((colv == tok_t) | (tok_t == MASK_ID))
        comb = cands[0]
        for bb in range(1, BS):
            comb = comb | cands[bb]

        def _slow(g=g, cnts=cnts):
            def _lbody(l, inner):
                vecs, cnts = inner
                n = g * L + l
                ds0 = ds_v[pl.ds(n * C, L)]
                new_vecs, new_cnts = [], []
                for bb in range(BS):
                    ok0 = (ds0 == in0[bb]) | wild0[bb]

                    def _full(bb=bb, n=n, ok0=ok0):
                        def _cbody(t, acc):
                            dsv = ds_v[pl.ds(n * C + t * L, L)]
                            inv = in_v[pl.ds(bb * C + t * L, L)]
                            return acc & ((dsv == inv) | (inv == MASK_ID))
                        acc = lax.fori_loop(1, CCHUNKS, _cbody, ok0)
                        return jnp.where(_all_lanes(acc), jnp.float32(1.0),
                                         jnp.float32(0.0))

                    flag = lax.cond(_all_lanes(ok0), _full,
                                    lambda: jnp.float32(0.0))
                    new_vecs.append(jnp.where(iota == l, flag, vecs[bb]))
                    new_cnts.append(cnts[bb] + flag)
                return tuple(new_vecs), tuple(new_cnts)

            vecs, cnts2 = lax.fori_loop(
                0, L, _lbody, (tuple(zero_vec for _ in range(BS)), cnts))
            return vecs + cnts2

        def _fast(cnts=cnts):
            return tuple(zero_vec for _ in range(BS)) + cnts

        res = lax.cond(_no_lanes(comb), _fast, _slow)
        vecs, cnts = res[:BS], res[BS:]
        for bb in range(BS):
            w_local[pl.ds(bb * NPT + g * L, L)] = vecs[bb]
        return cnts

    cnts = lax.fori_loop(0, NPT // L, _gbody,
                         tuple(jnp.float32(0.0) for _ in range(BS)))

    cv = zero_vec
    for bb in range(BS):
        cv = jnp.where(iota == bb, cnts[bb], cv)
    cnt_v[...] = cv

    pltpu.sync_copy(w_local, w_sh.at[pl.ds(sid * (BS * NPT), BS * NPT)])
    pltpu.sync_copy(cnt_v, cnt_sh.at[pl.ds(sid * L, L)])

    for k in range(1, NZ):
        _drain(k - 1, cp)
        cp = _fire(k)
    _drain(NZ - 1, cp)

    plsc.subcore_barrier()
    pltpu.sync_copy(cnt_sh, cnt_all)
    ns = cnt_all[pl.ds(0, L)]
    for s in range(1, NS):
        ns = ns + cnt_all[pl.ds(s * L, L)]
    ns_v[...] = ns

    my_ns = ns[BS - 1]
    for bb in range(BS - 1):
        my_ns = jnp.where(b == bb, ns[bb], my_ns)

    @pl.when(my_ns > 0.0)
    def _match():
        inv_v = ones_vec / jnp.full((L,), my_ns, jnp.float32)
        pltpu.sync_copy(w_sh, w_all)

        def _blk(blk, _):
            def _acc(j, _2):
                pltpu.sync_copy(ds_hbm.at[pl.ds(j * L * C, L * C)], dsrow_v)
                off = ((j // (NPT // L)) * (BS * NPT) + b * NPT
                       + (j % (NPT // L)) * L)
                msk = w_all[pl.ds(off, L)] > 0.5
                for r in range(ZROWS):
                    col = base_c + blk * ZROWS + r
                    toks = plsc.load_gather(dsrow_v, [iota * C + col])
                    plsc.addupdate_scatter(buf, [jnp.full((L,), r, jnp.int32),
                                                 toks], inv_v, mask=msk)
                return 0
            lax.fori_loop(0, N // L, _acc, 0)

            pltpu.sync_copy(buf,
                            out_hbm.at[pl.ds(row_base + blk * ZROWS, ZROWS)])

            def _z2(i, _2):
                for r in range(ZROWS):
                    buf[r, pl.ds(i * L, L)] = jnp.zeros((L,), jnp.float32)
                return 0
            lax.fori_loop(0, NUM_TOKENS // L, _z2, 0)
            return 0

        lax.fori_loop(0, NZ, _blk, 0)


@jax.jit
def _impl(input_tokens, dataset_tokens):
    mesh = plsc.VectorSubcoreMesh(core_axis_name="c", subcore_axis_name="s")
    run = functools.partial(
        pl.kernel,
        mesh=mesh,
        compiler_params=pltpu.CompilerParams(needs_layout_passes=False),
        out_type=jax.ShapeDtypeStruct((ROWS, NUM_TOKENS), jnp.float32),
        scratch_types=[
            pltpu.VMEM((BS * C + L,), jnp.int32),
            pltpu.VMEM((NPT * C,), jnp.int32),
            pltpu.VMEM((L * C,), jnp.int32),
            pltpu.VMEM((ZROWS, NUM_TOKENS), jnp.float32),
            pltpu.VMEM((BS * NPT,), jnp.float32),
            pltpu.VMEM((L,), jnp.float32),
            pltpu.VMEM((NS * BS * NPT,), jnp.float32),
            pltpu.VMEM((NS * L,), jnp.float32),
            pltpu.VMEM((L,), jnp.float32),
            pltpu.VMEM_SHARED((NS * BS * NPT,), jnp.float32),
            pltpu.VMEM_SHARED((NS * L,), jnp.float32),
            pltpu.SemaphoreType.DMA,
        ],
    )(_sc_body)
    out2d = run(input_tokens.reshape(-1), dataset_tokens.reshape(-1))
    return out2d.reshape(BS, C, NUM_TOKENS)


def kernel(input_tokens, dataset_tokens, t):
    del t
    return _impl(input_tokens, dataset_tokens)


# speedup vs baseline: 2.6949x; 1.0208x over previous
import functools

import jax
import jax.numpy as jnp
from jax import lax
from jax.experimental import pallas as pl
from jax.experimental.pallas import tpu as pltpu
from jax.experimental.pallas import tpu_sc as plsc

NUM_TOKENS = 8192
MASK_ID = 3
BS, C, N = 4, 384, 1024

L = 16
NC, NS = 2, 16
NW = NC * NS
ROWS = BS * C
RPW = ROWS // NW
WPB = NW // BS
NPT = N // NS
CCHUNKS = C // L
ZROWS = 8
NZ = RPW // ZROWS
SCREEN = 8


def _all_lanes(x):
    return plsc.all_reduce_population_count(x)[0] == L


def _no_lanes(x):
    return plsc.all_reduce_population_count(x)[0] == 0


def _sc_body(in_hbm, ds_hbm, out_hbm,
             in_v, ds_v, dsrow_v, buf, w_local, cnt_v, w_all,
             cnt_all, ns_v, w_sh, cnt_sh, sem):
    cid = lax.axis_index("c")
    sid = lax.axis_index("s")
    wid = cid * NS + sid
    iota = lax.iota(jnp.int32, L)

    pltpu.sync_copy(in_hbm, in_v.at[:, pl.ds(0, C)])
    pltpu.sync_copy(ds_hbm.at[pl.ds(sid * NPT, NPT)], ds_v)

    def _zloop(i, _):
        for r in range(ZROWS):
            buf[r, pl.ds(i * L, L)] = jnp.zeros((L,), jnp.float32)
        return 0
    lax.fori_loop(0, NUM_TOKENS // L, _zloop, 0)

    row_base = wid * RPW
    b = wid // WPB
    base_c = (wid % WPB) * RPW
    ones_vec = jnp.ones((L,), jnp.float32)
    zero_vec = jnp.zeros((L,), jnp.float32)
    punch_mask = iota < ZROWS

    def _holes(k):
        return in_v[b, pl.ds(base_c + k * ZROWS, L)]

    def _fire(k):
        plsc.store_scatter(buf, [iota, _holes(k)], ones_vec, mask=punch_mask)
        dst = out_hbm.at[pl.ds(row_base + k * ZROWS, ZROWS)]
        return pltpu.async_copy(buf, dst, sem)

    def _drain(k, cp):
        cp.wait()
        plsc.store_scatter(buf, [iota, _holes(k)], zero_vec, mask=punch_mask)

    cp = _fire(0)

    in0 = [in_v[bb, pl.ds(0, L)] for bb in range(BS)]
    wild0 = [v == MASK_ID for v in in0]

    def _gbody(g, cnts):
        nvec = g * L + iota
        cands = [iota == iota for _ in range(BS)]
        for t in range(SCREEN):
            colv = plsc.load_gather(ds_v, [nvec, jnp.full((L,), t, jnp.int32)])
            for bb in range(BS):
                tok_t = in0[bb][t]
                cands[bb] = cands[bb] & ((colv == tok_t) | (tok_t == MASK_ID))
        comb = cands[0]
        for bb in range(1, BS):
            comb = comb | cands[bb]

        def _slow(g=g, cnts=cnts):
            def _lbody(l, inner):
                vecs, cnts = inner
                n = g * L + l
                ds0 = ds_v[n, pl.ds(0, L)]
                new_vecs, new_cnts = [], []
                for bb in range(BS):
                    ok0 = (ds0 == in0[bb]) | wild0[bb]

                    def _full(bb=bb, n=n, ok0=ok0):
                        def _cbody(t, acc):
                            dsv = ds_v[n, pl.ds(t * L, L)]
                            inv = in_v[bb, pl.ds(t * L, L)]
                            return acc & ((dsv == inv) | (inv == MASK_ID))
                        acc = lax.fori_loop(1, CCHUNKS, _cbody, ok0)
                        return jnp.where(_all_lanes(acc), jnp.float32(1.0),
                                         jnp.float32(0.0))

                    flag = lax.cond(_all_lanes(ok0), _full,
                                    lambda: jnp.float32(0.0))
                    new_vecs.append(jnp.where(iota == l, flag, vecs[bb]))
                    new_cnts.append(cnts[bb] + flag)
                return tuple(new_vecs), tuple(new_cnts)

            vecs, cnts2 = lax.fori_loop(
                0, L, _lbody, (tuple(zero_vec for _ in range(BS)), cnts))
            return vecs + cnts2

        def _fast(cnts=cnts):
            return tuple(zero_vec for _ in range(BS)) + cnts

        res = lax.cond(_no_lanes(comb), _fast, _slow)
        vecs, cnts = res[:BS], res[BS:]
        for bb in range(BS):
            w_local[pl.ds(bb * NPT + g * L, L)] = vecs[bb]
        return cnts

    cnts = lax.fori_loop(0, NPT // L, _gbody,
                         tuple(jnp.float32(0.0) for _ in range(BS)))

    cv = zero_vec
    for bb in range(BS):
        cv = jnp.where(iota == bb, cnts[bb], cv)
    cnt_v[...] = cv

    pltpu.sync_copy(w_local, w_sh.at[pl.ds(sid * (BS * NPT), BS * NPT)])
    pltpu.sync_copy(cnt_v, cnt_sh.at[pl.ds(sid * L, L)])

    for k in range(1, NZ):
        _drain(k - 1, cp)
        cp = _fire(k)
    _drain(NZ - 1, cp)

    plsc.subcore_barrier()
    pltpu.sync_copy(cnt_sh, cnt_all)
    ns = cnt_all[pl.ds(0, L)]
    for s in range(1, NS):
        ns = ns + cnt_all[pl.ds(s * L, L)]
    ns_v[...] = ns

    my_ns = ns[BS - 1]
    for bb in range(BS - 1):
        my_ns = jnp.where(b == bb, ns[bb], my_ns)

    @pl.when(my_ns > 0.0)
    def _match():
        inv_v = ones_vec / jnp.full((L,), my_ns, jnp.float32)
        pltpu.sync_copy(w_sh, w_all)

        def _blk(blk, _):
            def _acc(j, _2):
                pltpu.sync_copy(ds_hbm.at[pl.ds(j * L, L)], dsrow_v)
                off = ((j // (NPT // L)) * (BS * NPT) + b * NPT
                       + (j % (NPT // L)) * L)
                msk = w_all[pl.ds(off, L)] > 0.5
                for r in range(ZROWS):
                    col = base_c + blk * ZROWS + r
                    toks = plsc.load_gather(
                        dsrow_v, [iota, jnp.full((L,), col, jnp.int32)])
                    plsc.addupdate_scatter(buf, [jnp.full((L,), r, jnp.int32),
                                                 toks], inv_v, mask=msk)
                return 0
            lax.fori_loop(0, N // L, _acc, 0)

            pltpu.sync_copy(buf,
                            out_hbm.at[pl.ds(row_base + blk * ZROWS, ZROWS)])

            def _z2(i, _2):
                for r in range(ZROWS):
                    buf[r, pl.ds(i * L, L)] = jnp.zeros((L,), jnp.float32)
                return 0
            lax.fori_loop(0, NUM_TOKENS // L, _z2, 0)
            return 0

        lax.fori_loop(0, NZ, _blk, 0)


@jax.jit
def _impl(input_tokens, dataset_tokens):
    mesh = plsc.VectorSubcoreMesh(core_axis_name="c", subcore_axis_name="s")
    run = functools.partial(
        pl.kernel,
        mesh=mesh,
        compiler_params=pltpu.CompilerParams(needs_layout_passes=False),
        out_type=jax.ShapeDtypeStruct((ROWS, NUM_TOKENS), jnp.float32),
        scratch_types=[
            pltpu.VMEM((BS, C + L), jnp.int32),
            pltpu.VMEM((NPT, C), jnp.int32),
            pltpu.VMEM((L, C), jnp.int32),
            pltpu.VMEM((ZROWS, NUM_TOKENS), jnp.float32),
            pltpu.VMEM((BS * NPT,), jnp.float32),
            pltpu.VMEM((L,), jnp.float32),
            pltpu.VMEM((NS * BS * NPT,), jnp.float32),
            pltpu.VMEM((NS * L,), jnp.float32),
            pltpu.VMEM((L,), jnp.float32),
            pltpu.VMEM_SHARED((NS * BS * NPT,), jnp.float32),
            pltpu.VMEM_SHARED((NS * L,), jnp.float32),
            pltpu.SemaphoreType.DMA,
        ],
    )(_sc_body)
    out2d = run(input_tokens, dataset_tokens)
    return out2d.reshape(BS, C, NUM_TOKENS)


def kernel(input_tokens, dataset_tokens, t):
    del t
    return _impl(input_tokens, dataset_tokens)


# speedup vs baseline: 2.7050x; 1.0037x over previous
import functools

import jax
import jax.numpy as jnp
from jax import lax
from jax.experimental import pallas as pl
from jax.experimental.pallas import tpu as pltpu
from jax.experimental.pallas import tpu_sc as plsc

NUM_TOKENS = 8192
MASK_ID = 3
BS, C, N = 4, 384, 1024

L = 16
NC, NS = 2, 16
NW = NC * NS
ROWS = BS * C
RPW = ROWS // NW
WPB = NW // BS
NPT = N // NS
CCHUNKS = C // L
ZROWS = 8
NZ = RPW // ZROWS
HW = 2048
NH = NUM_TOKENS // HW
HSH = 11
SCREEN = 8


def _all_lanes(x):
    return plsc.all_reduce_population_count(x)[0] == L


def _no_lanes(x):
    return plsc.all_reduce_population_count(x)[0] == 0


def _sc_body(in_hbm, ds_hbm, out_hbm,
             in_v, ds_v, dsrow_v, buf0, buf1, w_local, cnt_v, w_all,
             cnt_all, ns_v, w_sh, cnt_sh, sem0, sem1):
    cid = lax.axis_index("c")
    sid = lax.axis_index("s")
    wid = cid * NS + sid
    iota = lax.iota(jnp.int32, L)

    pltpu.sync_copy(in_hbm, in_v.at[:, pl.ds(0, C)])
    pltpu.sync_copy(ds_hbm.at[pl.ds(sid * NPT, NPT)], ds_v)

    def _zloop(i, _):
        for r in range(ZROWS):
            buf0[r, pl.ds(i * L, L)] = jnp.zeros((L,), jnp.float32)
            buf1[r, pl.ds(i * L, L)] = jnp.zeros((L,), jnp.float32)
        return 0
    lax.fori_loop(0, HW // L, _zloop, 0)

    row_base = wid * RPW
    b = wid // WPB
    base_c = (wid % WPB) * RPW
    ones_vec = jnp.ones((L,), jnp.float32)
    zero_vec = jnp.zeros((L,), jnp.float32)
    punch_mask = iota < ZROWS
    bufs = (buf0, buf1)
    sems = (sem0, sem1)

    def _holes(k):
        return in_v[b, pl.ds(base_c + k * ZROWS, L)]

    def _punch(q, val):
        k, h = q // NH, q % NH
        toks = _holes(k)
        msk = punch_mask & ((toks >> HSH) == h)
        plsc.store_scatter(bufs[q % 2], [iota, toks & (HW - 1)], val, mask=msk)

    def _fire(q):
        k, h = q // NH, q % NH
        _punch(q, ones_vec)
        dst = out_hbm.at[pl.ds(row_base + k * ZROWS, ZROWS),
                         pl.ds(h * HW, HW)]
        return pltpu.async_copy(bufs[q % 2], dst, sems[q % 2])

    def _drain(q, cp):
        cp.wait()
        _punch(q, zero_vec)

    zcps = [_fire(0), _fire(1)]

    in0 = [in_v[bb, pl.ds(0, L)] for bb in range(BS)]
    wild0 = [v == MASK_ID for v in in0]

    def _gbody(g, cnts):
        nvec = g * L + iota
        cands = [iota == iota for _ in range(BS)]
        for t in range(SCREEN):
            colv = plsc.load_gather(ds_v, [nvec, jnp.full((L,), t, jnp.int32)])
            for bb in range(BS):
                tok_t = in0[bb][t]
                cands[bb] = cands[bb] & ((colv == tok_t) | (tok_t == MASK_ID))
        comb = cands[0]
        for bb in range(1, BS):
            comb = comb | cands[bb]

        def _slow(g=g, cnts=cnts):
            def _lbody(l, inner):
                vecs, cnts = inner
                n = g * L + l
                ds0 = ds_v[n, pl.ds(0, L)]
                new_vecs, new_cnts = [], []
                for bb in range(BS):
                    ok0 = (ds0 == in0[bb]) | wild0[bb]

                    def _full(bb=bb, n=n, ok0=ok0):
                        def _cbody(t, acc):
                            dsv = ds_v[n, pl.ds(t * L, L)]
                            inv = in_v[bb, pl.ds(t * L, L)]
                            return acc & ((dsv == inv) | (inv == MASK_ID))
                        acc = lax.fori_loop(1, CCHUNKS, _cbody, ok0)
                        return jnp.where(_all_lanes(acc), jnp.float32(1.0),
                                         jnp.float32(0.0))

                    flag = lax.cond(_all_lanes(ok0), _full,
                                    lambda: jnp.float32(0.0))
                    new_vecs.append(jnp.where(iota == l, flag, vecs[bb]))
                    new_cnts.append(cnts[bb] + flag)
                return tuple(new_vecs), tuple(new_cnts)

            vecs, cnts2 = lax.fori_loop(
                0, L, _lbody, (tuple(zero_vec for _ in range(BS)), cnts))
            return vecs + cnts2

        def _fast(cnts=cnts):
            return tuple(zero_vec for _ in range(BS)) + cnts

        res = lax.cond(_no_lanes(comb), _fast, _slow)
        vecs, cnts = res[:BS], res[BS:]
        for bb in range(BS):
            w_local[pl.ds(bb * NPT + g * L, L)] = vecs[bb]
        return cnts

    cnts = lax.fori_loop(0, NPT // L, _gbody,
                         tuple(jnp.float32(0.0) for _ in range(BS)))

    cv = zero_vec
    for bb in range(BS):
        cv = jnp.where(iota == bb, cnts[bb], cv)
    cnt_v[...] = cv

    pltpu.sync_copy(w_local, w_sh.at[pl.ds(sid * (BS * NPT), BS * NPT)])
    pltpu.sync_copy(cnt_v, cnt_sh.at[pl.ds(sid * L, L)])

    for q in range(2, NZ * NH):
        _drain(q - 2, zcps[q - 2])
        zcps.append(_fire(q))
    _drain(NZ * NH - 2, zcps[NZ * NH - 2])
    _drain(NZ * NH - 1, zcps[NZ * NH - 1])

    plsc.subcore_barrier()
    pltpu.sync_copy(cnt_sh, cnt_all)
    ns = cnt_all[pl.ds(0, L)]
    for s in range(1, NS):
        ns = ns + cnt_all[pl.ds(s * L, L)]
    ns_v[...] = ns

    my_ns = ns[BS - 1]
    for bb in range(BS - 1):
        my_ns = jnp.where(b == bb, ns[bb], my_ns)

    @pl.when(my_ns > 0.0)
    def _match():
        inv_v = ones_vec / jnp.full((L,), my_ns, jnp.float32)
        pltpu.sync_copy(w_sh, w_all)

        def _stripe(q, _):
            blk, h = q // NH, q % NH

            def _z2(i, _2):
                for r in range(ZROWS):
                    buf0[r, pl.ds(i * L, L)] = jnp.zeros((L,), jnp.float32)
                return 0
            lax.fori_loop(0, HW // L, _z2, 0)

            def _acc(j, _2):
                pltpu.sync_copy(ds_hbm.at[pl.ds(j * L, L)], dsrow_v)
                off = ((j // (NPT // L)) * (BS * NPT) + b * NPT
                       + (j % (NPT // L)) * L)
                wmsk = w_all[pl.ds(off, L)] > 0.5
                for r in range(ZROWS):
                    col = base_c + blk * ZROWS + r
                    toks = plsc.load_gather(
                        dsrow_v, [iota, jnp.full((L,), col, jnp.int32)])
                    msk = wmsk & ((toks >> HSH) == h)
                    plsc.addupdate_scatter(
                        buf0, [jnp.full((L,), r, jnp.int32), toks & (HW - 1)],
                        inv_v, mask=msk)
                return 0
            lax.fori_loop(0, N // L, _acc, 0)

            pltpu.sync_copy(buf0,
                            out_hbm.at[pl.ds(row_base + blk * ZROWS, ZROWS),
                                       pl.ds(h * HW, HW)])
            return 0

        lax.fori_loop(0, NZ * NH, _stripe, 0)


@jax.jit
def _impl(input_tokens, dataset_tokens):
    mesh = plsc.VectorSubcoreMesh(core_axis_name="c", subcore_axis_name="s")
    run = functools.partial(
        pl.kernel,
        mesh=mesh,
        compiler_params=pltpu.CompilerParams(needs_layout_passes=False),
        out_type=jax.ShapeDtypeStruct((ROWS, NUM_TOKENS), jnp.float32),
        scratch_types=[
            pltpu.VMEM((BS, C + L), jnp.int32),
            pltpu.VMEM((NPT, C), jnp.int32),
            pltpu.VMEM((L, C), jnp.int32),
            pltpu.VMEM((ZROWS, HW), jnp.float32),
            pltpu.VMEM((ZROWS, HW), jnp.float32),
            pltpu.VMEM((BS * NPT,), jnp.float32),
            pltpu.VMEM((L,), jnp.float32),
            pltpu.VMEM((NS * BS * NPT,), jnp.float32),
            pltpu.VMEM((NS * L,), jnp.float32),
            pltpu.VMEM((L,), jnp.float32),
            pltpu.VMEM_SHARED((NS * BS * NPT,), jnp.float32),
            pltpu.VMEM_SHARED((NS * L,), jnp.float32),
            pltpu.SemaphoreType.DMA,
            pltpu.SemaphoreType.DMA,
        ],
    )(_sc_body)
    out2d = run(input_tokens, dataset_tokens)
    return out2d.reshape(BS, C, NUM_TOKENS)


def kernel(input_tokens, dataset_tokens, t):
    del t
    return _impl(input_tokens, dataset_tokens)


# speedup vs baseline: 2.7581x; 1.0196x over previous
import functools

import jax
import jax.numpy as jnp
from jax import lax
from jax.experimental import pallas as pl
from jax.experimental.pallas import tpu as pltpu
from jax.experimental.pallas import tpu_sc as plsc

NUM_TOKENS = 8192
MASK_ID = 3
BS, C, N = 4, 384, 1024

L = 16
NC, NS = 2, 16
NW = NC * NS
ROWS = BS * C
RPW = ROWS // NW
WPB = NW // BS
NPT = N // NS
CCHUNKS = C // L
ZROWS = 8
NZ = RPW // ZROWS
HW = 2048
NH = NUM_TOKENS // HW
HSH = 11
SCREEN = 8


def _all_lanes(x):
    return plsc.all_reduce_population_count(x)[0] == L


def _no_lanes(x):
    return plsc.all_reduce_population_count(x)[0] == 0


def _sc_body(in_hbm, ds_hbm, out_hbm,
             in_v, ds_v, dsrow_v, buf0, buf1, w_local, cnt_v, w_all,
             cnt_all, ns_v, w_sh, cnt_sh, sem0, sem1, dsem):
    cid = lax.axis_index("c")
    sid = lax.axis_index("s")
    wid = cid * NS + sid
    iota = lax.iota(jnp.int32, L)

    ds_cp = pltpu.async_copy(ds_hbm.at[pl.ds(sid * NPT, NPT)], ds_v, dsem)
    pltpu.sync_copy(in_hbm, in_v.at[:, pl.ds(0, C)])

    def _zloop(i, _):
        for r in range(ZROWS):
            buf0[r, pl.ds(i * L, L)] = jnp.zeros((L,), jnp.float32)
            buf1[r, pl.ds(i * L, L)] = jnp.zeros((L,), jnp.float32)
        return 0
    lax.fori_loop(0, HW // L, _zloop, 0)

    row_base = wid * RPW
    b = wid // WPB
    base_c = (wid % WPB) * RPW
    ones_vec = jnp.ones((L,), jnp.float32)
    zero_vec = jnp.zeros((L,), jnp.float32)
    punch_mask = iota < ZROWS
    bufs = (buf0, buf1)
    sems = (sem0, sem1)

    def _holes(k):
        return in_v[b, pl.ds(base_c + k * ZROWS, L)]

    def _punch(q, val):
        k, h = q // NH, q % NH
        toks = _holes(k)
        msk = punch_mask & ((toks >> HSH) == h)
        plsc.store_scatter(bufs[q % 2], [iota, toks & (HW - 1)], val, mask=msk)

    def _fire(q):
        k, h = q // NH, q % NH
        _punch(q, ones_vec)
        dst = out_hbm.at[pl.ds(row_base + k * ZROWS, ZROWS),
                         pl.ds(h * HW, HW)]
        return pltpu.async_copy(bufs[q % 2], dst, sems[q % 2])

    def _drain(q, cp):
        cp.wait()
        _punch(q, zero_vec)

    zcps = [_fire(0), _fire(1)]

    ds_cp.wait()
    in0 = [in_v[bb, pl.ds(0, L)] for bb in range(BS)]
    wild0 = [v == MASK_ID for v in in0]

    def _gbody(g, cnts):
        nvec = g * L + iota
        cands = [iota == iota for _ in range(BS)]
        for t in range(SCREEN):
            colv = plsc.load_gather(ds_v, [nvec, jnp.full((L,), t, jnp.int32)])
            for bb in range(BS):
                tok_t = in0[bb][t]
                cands[bb] = cands[bb] & ((colv == tok_t) | (tok_t == MASK_ID))
        comb = cands[0]
        for bb in range(1, BS):
            comb = comb | cands[bb]

        def _slow(g=g, cnts=cnts):
            def _lbody(l, inner):
                vecs, cnts = inner
                n = g * L + l
                ds0 = ds_v[n, pl.ds(0, L)]
                new_vecs, new_cnts = [], []
                for bb in range(BS):
                    ok0 = (ds0 == in0[bb]) | wild0[bb]

                    def _full(bb=bb, n=n, ok0=ok0):
                        def _cbody(t, acc):
                            dsv = ds_v[n, pl.ds(t * L, L)]
                            inv = in_v[bb, pl.ds(t * L, L)]
                            return acc & ((dsv == inv) | (inv == MASK_ID))
                        acc = lax.fori_loop(1, CCHUNKS, _cbody, ok0)
                        return jnp.where(_all_lanes(acc), jnp.float32(1.0),
                                         jnp.float32(0.0))

                    flag = lax.cond(_all_lanes(ok0), _full,
                                    lambda: jnp.float32(0.0))
                    new_vecs.append(jnp.where(iota == l, flag, vecs[bb]))
                    new_cnts.append(cnts[bb] + flag)
                return tuple(new_vecs), tuple(new_cnts)

            vecs, cnts2 = lax.fori_loop(
                0, L, _lbody, (tuple(zero_vec for _ in range(BS)), cnts))
            return vecs + cnts2

        def _fast(cnts=cnts):
            return tuple(zero_vec for _ in range(BS)) + cnts

        res = lax.cond(_no_lanes(comb), _fast, _slow)
        vecs, cnts = res[:BS], res[BS:]
        for bb in range(BS):
            w_local[pl.ds(bb * NPT + g * L, L)] = vecs[bb]
        return cnts

    cnts = lax.fori_loop(0, NPT // L, _gbody,
                         tuple(jnp.float32(0.0) for _ in range(BS)))

    cv = zero_vec
    for bb in range(BS):
        cv = jnp.where(iota == bb, cnts[bb], cv)
    cnt_v[...] = cv

    for q in range(2, NZ * NH):
        _drain(q - 2, zcps[q - 2])
        zcps.append(_fire(q))
    _drain(NZ * NH - 2, zcps[NZ * NH - 2])
    _drain(NZ * NH - 1, zcps[NZ * NH - 1])

    pltpu.sync_copy(w_local, w_sh.at[pl.ds(sid * (BS * NPT), BS * NPT)])
    pltpu.sync_copy(cnt_v, cnt_sh.at[pl.ds(sid * L, L)])

    plsc.subcore_barrier()
    pltpu.sync_copy(cnt_sh, cnt_all)
    ns = cnt_all[pl.ds(0, L)]
    for s in range(1, NS):
        ns = ns + cnt_all[pl.ds(s * L, L)]
    ns_v[...] = ns

    my_ns = ns[BS - 1]
    for bb in range(BS - 1):
        my_ns = jnp.where(b == bb, ns[bb], my_ns)

    @pl.when(my_ns > 0.0)
    def _match():
        inv_v = ones_vec / jnp.full((L,), my_ns, jnp.float32)
        pltpu.sync_copy(w_sh, w_all)

        def _stripe(q, _):
            blk, h = q // NH, q % NH

            def _z2(i, _2):
                for r in range(ZROWS):
                    buf0[r, pl.ds(i * L, L)] = jnp.zeros((L,), jnp.float32)
                return 0
            lax.fori_loop(0, HW // L, _z2, 0)

            def _acc(j, _2):
                pltpu.sync_copy(ds_hbm.at[pl.ds(j * L, L)], dsrow_v)
                off = ((j // (NPT // L)) * (BS * NPT) + b * NPT
                       + (j % (NPT // L)) * L)
                wmsk = w_all[pl.ds(off, L)] > 0.5
                for r in range(ZROWS):
                    col = base_c + blk * ZROWS + r
                    toks = plsc.load_gather(
                        dsrow_v, [iota, jnp.full((L,), col, jnp.int32)])
                    msk = wmsk & ((toks >> HSH) == h)
                    plsc.addupdate_scatter(
                        buf0, [jnp.full((L,), r, jnp.int32), toks & (HW - 1)],
                        inv_v, mask=msk)
                return 0
            lax.fori_loop(0, N // L, _acc, 0)

            pltpu.sync_copy(buf0,
                            out_hbm.at[pl.ds(row_base + blk * ZROWS, ZROWS),
                                       pl.ds(h * HW, HW)])
            return 0

        lax.fori_loop(0, NZ * NH, _stripe, 0)


@jax.jit
def _impl(input_tokens, dataset_tokens):
    mesh = plsc.VectorSubcoreMesh(core_axis_name="c", subcore_axis_name="s")
    run = functools.partial(
        pl.kernel,
        mesh=mesh,
        compiler_params=pltpu.CompilerParams(needs_layout_passes=False),
        out_type=jax.ShapeDtypeStruct((ROWS, NUM_TOKENS), jnp.float32),
        scratch_types=[
            pltpu.VMEM((BS, C + L), jnp.int32),
            pltpu.VMEM((NPT, C), jnp.int32),
            pltpu.VMEM((L, C), jnp.int32),
            pltpu.VMEM((ZROWS, HW), jnp.float32),
            pltpu.VMEM((ZROWS, HW), jnp.float32),
            pltpu.VMEM((BS * NPT,), jnp.float32),
            pltpu.VMEM((L,), jnp.float32),
            pltpu.VMEM((NS * BS * NPT,), jnp.float32),
            pltpu.VMEM((NS * L,), jnp.float32),
            pltpu.VMEM((L,), jnp.float32),
            pltpu.VMEM_SHARED((NS * BS * NPT,), jnp.float32),
            pltpu.VMEM_SHARED((NS * L,), jnp.float32),
            pltpu.SemaphoreType.DMA,
            pltpu.SemaphoreType.DMA,
            pltpu.SemaphoreType.DMA,
        ],
    )(_sc_body)
    out2d = run(input_tokens, dataset_tokens)
    return out2d.reshape(BS, C, NUM_TOKENS)


def kernel(input_tokens, dataset_tokens, t):
    del t
    return _impl(input_tokens, dataset_tokens)


# speedup vs baseline: 2.8131x; 1.0199x over previous
import functools

import jax
import jax.numpy as jnp
from jax import lax
from jax.experimental import pallas as pl
from jax.experimental.pallas import tpu as pltpu
from jax.experimental.pallas import tpu_sc as plsc

NUM_TOKENS = 8192
MASK_ID = 3
BS, C, N = 4, 384, 1024

L = 16
NC, NS = 2, 16
NW = NC * NS
ROWS = BS * C
RPW = ROWS // NW
WPB = NW // BS
NPT = N // NS
CCHUNKS = C // L
ZROWS = 8
NZ = RPW // ZROWS
HW = 2048
NH = NUM_TOKENS // HW
HSH = 11
SCREEN = 8


def _all_lanes(x):
    return plsc.all_reduce_population_count(x)[0] == L


def _no_lanes(x):
    return plsc.all_reduce_population_count(x)[0] == 0


def _sc_body(in_hbm, ds_hbm, out_hbm,
             in_v, ds_v, dsrow_v, buf0, buf1, w_local, cnt_v, w_all,
             cnt_all, ns_v, w_sh, cnt_sh, sem0, sem1, dsem):
    cid = lax.axis_index("c")
    sid = lax.axis_index("s")
    wid = cid * NS + sid
    iota = lax.iota(jnp.int32, L)

    ds_cp = pltpu.async_copy(ds_hbm.at[pl.ds(sid * NPT, NPT)], ds_v, dsem)
    pltpu.sync_copy(in_hbm, in_v.at[:, pl.ds(0, C)])

    def _zloop(i, _):
        for r in range(ZROWS):
            buf0[r, pl.ds(i * L, L)] = jnp.zeros((L,), jnp.float32)
            buf1[r, pl.ds(i * L, L)] = jnp.zeros((L,), jnp.float32)
        return 0
    lax.fori_loop(0, HW // L, _zloop, 0)

    row_base = wid * RPW
    b = wid // WPB
    base_c = (wid % WPB) * RPW
    ones_vec = jnp.ones((L,), jnp.float32)
    zero_vec = jnp.zeros((L,), jnp.float32)
    punch_mask = iota < ZROWS
    bufs = (buf0, buf1)
    sems = (sem0, sem1)

    def _holes(k):
        return in_v[b, pl.ds(base_c + k * ZROWS, L)]

    def _punch(q, val, j):
        k, h = q // NH, q % NH
        toks = _holes(k)
        msk = punch_mask & ((toks >> HSH) == h)
        plsc.store_scatter(bufs[j], [iota, toks & (HW - 1)], val, mask=msk)

    def _fire(q, j):
        k, h = q // NH, q % NH
        _punch(q, ones_vec, j)
        dst = out_hbm.at[pl.ds(row_base + k * ZROWS, ZROWS),
                         pl.ds(h * HW, HW)]
        return pltpu.async_copy(bufs[j], dst, sems[j])

    _fire(0, 0)
    _fire(1, 1)

    ds_cp.wait()
    in0 = [in_v[bb, pl.ds(0, L)] for bb in range(BS)]
    wild0 = [v == MASK_ID for v in in0]

    def _gbody(g, cnts):
        nvec = g * L + iota
        cands = [iota == iota for _ in range(BS)]
        for t in range(SCREEN):
            colv = plsc.load_gather(ds_v, [nvec, jnp.full((L,), t, jnp.int32)])
            for bb in range(BS):
                tok_t = in0[bb][t]
                cands[bb] = cands[bb] & ((colv == tok_t) | (tok_t == MASK_ID))
        comb = cands[0]
        for bb in range(1, BS):
            comb = comb | cands[bb]

        def _slow(g=g, cnts=cnts):
            def _lbody(l, inner):
                vecs, cnts = inner
                n = g * L + l
                ds0 = ds_v[n, pl.ds(0, L)]
                new_vecs, new_cnts = [], []
                for bb in range(BS):
                    ok0 = (ds0 == in0[bb]) | wild0[bb]

                    def _full(bb=bb, n=n, ok0=ok0):
                        def _cbody(t, acc):
                            dsv = ds_v[n, pl.ds(t * L, L)]
                            inv = in_v[bb, pl.ds(t * L, L)]
                            return acc & ((dsv == inv) | (inv == MASK_ID))
                        acc = lax.fori_loop(1, CCHUNKS, _cbody, ok0)
                        return jnp.where(_all_lanes(acc), jnp.float32(1.0),
                                         jnp.float32(0.0))

                    flag = lax.cond(_all_lanes(ok0), _full,
                                    lambda: jnp.float32(0.0))
                    new_vecs.append(jnp.where(iota == l, flag, vecs[bb]))
                    new_cnts.append(cnts[bb] + flag)
                return tuple(new_vecs), tuple(new_cnts)

            vecs, cnts2 = lax.fori_loop(
                0, L, _lbody, (tuple(zero_vec for _ in range(BS)), cnts))
            return vecs + cnts2

        def _fast(cnts=cnts):
            return tuple(zero_vec for _ in range(BS)) + cnts

        res = lax.cond(_no_lanes(comb), _fast, _slow)
        vecs, cnts = res[:BS], res[BS:]
        for bb in range(BS):
            w_local[pl.ds(bb * NPT + g * L, L)] = vecs[bb]
        return cnts

    cnts = lax.fori_loop(0, NPT // L, _gbody,
                         tuple(jnp.float32(0.0) for _ in range(BS)))

    cv = zero_vec
    for bb in range(BS):
        cv = jnp.where(iota == bb, cnts[bb], cv)
    cnt_v[...] = cv

    def _wait_pair(j):
        dst = out_hbm.at[pl.ds(row_base, ZROWS), pl.ds(0, HW)]
        pltpu.make_async_copy(bufs[j], dst, sems[j]).wait()

    def _qbody(i, _):
        for j in range(2):
            q = 2 * i + j
            _wait_pair(j)
            _punch(q - 2, zero_vec, j)
            _punch(q, ones_vec, j)
            k, h = q // NH, q % NH
            dst = out_hbm.at[pl.ds(row_base + k * ZROWS, ZROWS),
                             pl.ds(h * HW, HW)]
            pltpu.async_copy(bufs[j], dst, sems[j])
        return 0
    lax.fori_loop(1, (NZ * NH) // 2, _qbody, 0)
    for j in range(2):
        q = NZ * NH - 2 + j
        _wait_pair(j)
        _punch(q, zero_vec, j)

    pltpu.sync_copy(w_local, w_sh.at[pl.ds(sid * (BS * NPT), BS * NPT)])
    pltpu.sync_copy(cnt_v, cnt_sh.at[pl.ds(sid * L, L)])

    plsc.subcore_barrier()
    pltpu.sync_copy(cnt_sh, cnt_all)
    ns = cnt_all[pl.ds(0, L)]
    for s in range(1, NS):
        ns = ns + cnt_all[pl.ds(s * L, L)]
    ns_v[...] = ns

    my_ns = ns[BS - 1]
    for bb in range(BS - 1):
        my_ns = jnp.where(b == bb, ns[bb], my_ns)

    @pl.when(my_ns > 0.0)
    def _match():
        inv_v = ones_vec / jnp.full((L,), my_ns, jnp.float32)
        pltpu.sync_copy(w_sh, w_all)

        def _stripe(q, _):
            blk, h = q // NH, q % NH

            def _z2(i, _2):
                for r in range(ZROWS):
                    buf0[r, pl.ds(i * L, L)] = jnp.zeros((L,), jnp.float32)
                return 0
            lax.fori_loop(0, HW // L, _z2, 0)

            def _acc(j, _2):
                pltpu.sync_copy(ds_hbm.at[pl.ds(j * L, L)], dsrow_v)
                off = ((j // (NPT // L)) * (BS * NPT) + b * NPT
                       + (j % (NPT // L)) * L)
                wmsk = w_all[pl.ds(off, L)] > 0.5
                for r in range(ZROWS):
                    col = base_c + blk * ZROWS + r
                    toks = plsc.load_gather(
                        dsrow_v, [iota, jnp.full((L,), col, jnp.int32)])
                    msk = wmsk & ((toks >> HSH) == h)
                    plsc.addupdate_scatter(
                        buf0, [jnp.full((L,), r, jnp.int32), toks & (HW - 1)],
                        inv_v, mask=msk)
                return 0
            lax.fori_loop(0, N // L, _acc, 0)

            pltpu.sync_copy(buf0,
                            out_hbm.at[pl.ds(row_base + blk * ZROWS, ZROWS),
                                       pl.ds(h * HW, HW)])
            return 0

        lax.fori_loop(0, NZ * NH, _stripe, 0)


@jax.jit
def _impl(input_tokens, dataset_tokens):
    mesh = plsc.VectorSubcoreMesh(core_axis_name="c", subcore_axis_name="s")
    run = functools.partial(
        pl.kernel,
        mesh=mesh,
        compiler_params=pltpu.CompilerParams(needs_layout_passes=False),
        out_type=jax.ShapeDtypeStruct((ROWS, NUM_TOKENS), jnp.float32),
        scratch_types=[
            pltpu.VMEM((BS, C + L), jnp.int32),
            pltpu.VMEM((NPT, C), jnp.int32),
            pltpu.VMEM((L, C), jnp.int32),
            pltpu.VMEM((ZROWS, HW), jnp.float32),
            pltpu.VMEM((ZROWS, HW), jnp.float32),
            pltpu.VMEM((BS * NPT,), jnp.float32),
            pltpu.VMEM((L,), jnp.float32),
            pltpu.VMEM((NS * BS * NPT,), jnp.float32),
            pltpu.VMEM((NS * L,), jnp.float32),
            pltpu.VMEM((L,), jnp.float32),
            pltpu.VMEM_SHARED((NS * BS * NPT,), jnp.float32),
            pltpu.VMEM_SHARED((NS * L,), jnp.float32),
            pltpu.SemaphoreType.DMA,
            pltpu.SemaphoreType.DMA,
            pltpu.SemaphoreType.DMA,
        ],
    )(_sc_body)
    out2d = run(input_tokens, dataset_tokens)
    return out2d.reshape(BS, C, NUM_TOKENS)


def kernel(input_tokens, dataset_tokens, t):
    del t
    return _impl(input_tokens, dataset_tokens)
